# baseline (speedup 1.0000x reference)
import jax
import jax.numpy as jnp
from jax import lax
from jax.experimental import pallas as pl
from jax.experimental.pallas import tpu as pltpu
from jax.experimental.pallas import tpu_sc as plsc

_N = 32768
_B = 128
_K = 3277
_NK = _N - _K
_TD = _K / _N
_BOOST_STRENGTH = 1.0
_HBITS = 13
_HB = 1 << _HBITS
_RSHIFT = 32 - _HBITS
_RMASK = (1 << _RSHIFT) - 1
_CAP = 4096
_L = 16
_NTILES = 32
_RPT = _B // _NTILES


def _ukey(xv, bfv):
    b = xv * bfv
    u = lax.bitcast_convert_type(b, jnp.uint32)
    return jnp.where((u >> 31) != 0, ~u, u | jnp.uint32(0x80000000))


def _body(x_hbm, dc_hbm, o_hbm, bf_v, x_v, hist_v, ck_v, ci_v, cx_v):
    wid = lax.axis_index("s") * 2 + lax.axis_index("c")
    iota = lax.iota(jnp.int32, _L)
    ones = jnp.ones((_L,), jnp.int32)

    pltpu.sync_copy(dc_hbm, bf_v)

    def bf_body(i, c):
        sl = pl.ds(i * _L, _L)
        bf_v[sl] = jnp.exp(
            (jnp.float32(_TD) - bf_v[sl]) * jnp.float32(_BOOST_STRENGTH))
        return c

    lax.fori_loop(0, _N // _L, bf_body, 0)

    def row_body(r, c):
        row = wid * _RPT + r
        pltpu.sync_copy(x_hbm.at[row], x_v)

        def z_body(i, cz):
            hist_v[pl.ds(i * _L, _L)] = jnp.zeros((_L,), jnp.int32)
            return cz

        lax.fori_loop(0, _HB // _L, z_body, 0)

        def h_body(i, ch):
            sl = pl.ds(i * _L, _L)
            uk = _ukey(x_v[sl], bf_v[sl])
            bucket = (uk >> _RSHIFT).astype(jnp.int32)
            plsc.addupdate_scatter(hist_v, [bucket], ones)
            return ch

        lax.fori_loop(0, _N // _L, h_body, 0)

        def s_body(i, carry):
            nb, cb, tot = carry
            v = hist_v[pl.ds(i * _L, _L)]
            s = plsc.cumsum(v) + tot
            mask = s <= _NK
            nb = nb + jnp.max(plsc.all_reduce_population_count(mask))
            cb = jnp.maximum(cb, jnp.max(jnp.where(mask, s, 0)))
            tot = jnp.max(s)
            return nb, cb, tot

        b_star, c_b0, _tot = lax.fori_loop(
            0, _HB // _L, s_body, (jnp.int32(0), jnp.int32(0), jnp.int32(0)))

        def p2_body(i, wptr):
            sl = pl.ds(i * _L, _L)
            xv = x_v[sl]
            uk = _ukey(xv, bf_v[sl])
            bucket = (uk >> _RSHIFT).astype(jnp.int32)
            win = bucket > b_star
            x_v[sl] = jnp.where(win, xv, jnp.float32(0.0))
            cand = bucket == b_star
            pos = plsc.cumsum(jnp.where(cand, 1, 0))
            dst = pos + (wptr - 1)
            rres = (uk & jnp.uint32(_RMASK)).astype(jnp.int32)
            plsc.store_scatter(ck_v, [dst], rres, mask=cand)
            plsc.store_scatter(ci_v, [dst], iota + i * _L, mask=cand)
            plsc.store_scatter(cx_v, [dst], xv, mask=cand)
            return wptr + jnp.max(pos)

        m = lax.fori_loop(0, _N // _L, p2_body, jnp.int32(0))

        above = jnp.int32(_N) - c_b0 - m
        kp = jnp.int32(_K) - above
        nc = (m + _L - 1) // _L

        def bis_body(_, carry):
            lo, hi = carry
            mid = (lo + hi + 1) >> 1

            def cnt_body(j, acc):
                sl = pl.ds(j * _L, _L)
                rk = ck_v[sl]
                mm = ((iota + j * _L) < m) & (rk >= mid)
                return acc + plsc.all_reduce_population_count(mm)

            cntv = lax.fori_loop(0, nc, cnt_body, jnp.zeros((_L,), jnp.int32))
            pred = jnp.max(cntv) >= kp
            return jnp.where(pred, mid, lo), jnp.where(pred, hi, mid - 1)

        t_res, _hi = lax.fori_loop(
            0, _RSHIFT, bis_body, (jnp.int32(0), jnp.int32(_RMASK)))

        def f_body(j, cf):
            sl = pl.ds(j * _L, _L)
            wmask = ((iota + j * _L) < m) & (ck_v[sl] >= t_res)
            plsc.store_scatter(x_v, [ci_v[sl]], cx_v[sl], mask=wmask)
            return cf

        lax.fori_loop(0, nc, f_body, 0)

        pltpu.sync_copy(x_v, o_hbm.at[row])
        return c

    lax.fori_loop(0, _RPT, row_body, 0)


@jax.jit
def kernel(x, duty_cycles):
    run = pl.kernel(
        _body,
        out_type=jax.ShapeDtypeStruct((_B, _N), jnp.float32),
        mesh=plsc.VectorSubcoreMesh(core_axis_name="c", subcore_axis_name="s"),
        compiler_params=pltpu.CompilerParams(needs_layout_passes=False),
        scratch_types=[
            pltpu.VMEM((_N,), jnp.float32),
            pltpu.VMEM((_N,), jnp.float32),
            pltpu.VMEM((_HB,), jnp.int32),
            pltpu.VMEM((_CAP,), jnp.int32),
            pltpu.VMEM((_CAP,), jnp.int32),
            pltpu.VMEM((_CAP,), jnp.float32),
        ],
    )
    return run(x, duty_cycles)


# speedup vs baseline: 3.5182x; 3.5182x over previous
import jax
import jax.numpy as jnp
from jax import lax
from jax.experimental import pallas as pl
from jax.experimental.pallas import tpu as pltpu
from jax.experimental.pallas import tpu_sc as plsc

_N = 32768
_B = 128
_K = 3277
_NK = _N - _K
_TD = _K / _N
_BOOST_STRENGTH = 1.0
_HBITS = 13
_HB = 1 << _HBITS
_RSHIFT = 32 - _HBITS
_RMASK = (1 << _RSHIFT) - 1
_CAP = 4096
_L = 16
_NTILES = 32
_RPT = _B // _NTILES
_NBLK = 32
_CPB = _HB // _NBLK // _L


def _ukey(xv, bfv):
    b = xv * bfv
    u = lax.bitcast_convert_type(b, jnp.uint32)
    return jnp.where((u >> 31) != 0, ~u, u | jnp.uint32(0x80000000))


def _body(x_hbm, dc_hbm, o_hbm, bf_v, x_v, hist_v, ps_v, ck_v, ci_v, cx_v):
    wid = lax.axis_index("s") * 2 + lax.axis_index("c")
    iota = lax.iota(jnp.int32, _L)
    ones = jnp.ones((_L,), jnp.int32)
    zeros = jnp.zeros((_L,), jnp.int32)

    pltpu.sync_copy(dc_hbm, bf_v)

    @plsc.parallel_loop(0, _N // _L, unroll=8)
    def _(i):
        sl = pl.ds(i * _L, _L)
        bf_v[sl] = jnp.exp(
            (jnp.float32(_TD) - bf_v[sl]) * jnp.float32(_BOOST_STRENGTH))

    def row_body(r, c):
        row = wid * _RPT + r
        pltpu.sync_copy(x_hbm.at[row], x_v)

        @plsc.parallel_loop(0, _HB // _L, unroll=8)
        def _(i):
            hist_v[pl.ds(i * _L, _L)] = zeros

        @plsc.parallel_loop(0, _N // _L, unroll=8)
        def _(i):
            sl = pl.ds(i * _L, _L)
            uk = _ukey(x_v[sl], bf_v[sl])
            bucket = (uk >> _RSHIFT).astype(jnp.int32)
            plsc.addupdate_scatter(hist_v, [bucket], ones)

        @plsc.parallel_loop(0, _NBLK)
        def _(t):
            acc = zeros
            for u in range(_CPB):
                acc = acc + hist_v[pl.ds(t * (_CPB * _L) + u * _L, _L)]
            ps_v[pl.ds(t * _L, _L)] = acc

        def b_body(t, carry):
            pfx, nblk, base = carry
            tot = jnp.sum(ps_v[pl.ds(t * _L, _L)])
            pfx = pfx + tot
            ok = pfx <= _NK
            return (pfx, nblk + ok.astype(jnp.int32),
                    jnp.where(ok, pfx, base))

        _pfx, tstar, base = lax.fori_loop(
            0, _NBLK, b_body, (jnp.int32(0), jnp.int32(0), jnp.int32(0)))

        def c_body(ci, carry):
            nb, cbv, tot = carry
            v = hist_v[pl.ds(tstar * (_CPB * _L) + ci * _L, _L)]
            s = plsc.cumsum(v) + tot
            mask = s <= _NK
            nb = nb + plsc.all_reduce_population_count(mask)
            cbv = jnp.maximum(cbv, jnp.where(mask, s, 0))
            return nb, cbv, jnp.max(s)

        nbv, cbv, _tot = lax.fori_loop(0, _CPB, c_body, (zeros, zeros, base))
        b_star = tstar * (_CPB * _L) + jnp.max(nbv)
        c_b0 = jnp.maximum(jnp.max(cbv), base)

        @plsc.parallel_loop(0, _N // _L, unroll=4, carry=zeros)
        def wptr_v(i, w):
            sl = pl.ds(i * _L, _L)
            xv = x_v[sl]
            uk = _ukey(xv, bf_v[sl])
            bucket = (uk >> _RSHIFT).astype(jnp.int32)
            win = bucket > b_star
            x_v[sl] = jnp.where(win, xv, jnp.float32(0.0))
            cand = bucket == b_star
            pos = plsc.cumsum(jnp.where(cand, 1, 0))
            dst = pos + (w - 1)
            rres = (uk & jnp.uint32(_RMASK)).astype(jnp.int32)
            plsc.store_scatter(ck_v, [dst], rres, mask=cand)
            plsc.store_scatter(ci_v, [dst], iota + i * _L, mask=cand)
            plsc.store_scatter(cx_v, [dst], xv, mask=cand)
            return w + plsc.all_reduce_population_count(cand)

        m = jnp.max(wptr_v)
        above = jnp.int32(_N) - c_b0 - m
        kp = jnp.int32(_K) - above
        nc = (m + _L - 1) // _L

        def bis_body(_, carry):
            lo, hi = carry
            mid = (lo + hi + 1) >> 1

            def cnt_body(j, acc):
                sl = pl.ds(j * _L, _L)
                mm = ((iota + j * _L) < m) & (ck_v[sl] >= mid)
                return acc + plsc.all_reduce_population_count(mm)

            cntv = lax.fori_loop(0, nc, cnt_body, zeros)
            pred = jnp.max(cntv) >= kp
            return jnp.where(pred, mid, lo), jnp.where(pred, hi, mid - 1)

        t_res, _hi = lax.fori_loop(
            0, _RSHIFT, bis_body, (jnp.int32(0), jnp.int32(_RMASK)))

        def f_body(j, cf):
            sl = pl.ds(j * _L, _L)
            wmask = ((iota + j * _L) < m) & (ck_v[sl] >= t_res)
            plsc.store_scatter(x_v, [ci_v[sl]], cx_v[sl], mask=wmask)
            return cf

        lax.fori_loop(0, nc, f_body, 0)

        pltpu.sync_copy(x_v, o_hbm.at[row])
        return c

    lax.fori_loop(0, _RPT, row_body, 0)


@jax.jit
def kernel(x, duty_cycles):
    run = pl.kernel(
        _body,
        out_type=jax.ShapeDtypeStruct((_B, _N), jnp.float32),
        mesh=plsc.VectorSubcoreMesh(core_axis_name="c", subcore_axis_name="s"),
        compiler_params=pltpu.CompilerParams(needs_layout_passes=False),
        scratch_types=[
            pltpu.VMEM((_N,), jnp.float32),
            pltpu.VMEM((_N,), jnp.float32),
            pltpu.VMEM((_HB,), jnp.int32),
            pltpu.VMEM((_NBLK * _L,), jnp.int32),
            pltpu.VMEM((_CAP,), jnp.int32),
            pltpu.VMEM((_CAP,), jnp.int32),
            pltpu.VMEM((_CAP,), jnp.float32),
        ],
    )
    return run(x, duty_cycles)


# speedup vs baseline: 3.7904x; 1.0774x over previous
import jax
import jax.numpy as jnp
from jax import lax
from jax.experimental import pallas as pl
from jax.experimental.pallas import tpu as pltpu
from jax.experimental.pallas import tpu_sc as plsc

_N = 32768
_B = 128
_K = 3277
_NK = _N - _K
_TD = _K / _N
_BOOST_STRENGTH = 1.0
_HBITS = 13
_HB = 1 << _HBITS
_RSHIFT = 32 - _HBITS
_RMASK = (1 << _RSHIFT) - 1
_CAP = 4096
_L = 16
_NTILES = 32
_RPT = _B // _NTILES
_NBLK = 32
_CPB = _HB // _NBLK // _L


def _ukey(xv, bfv):
    b = xv * bfv
    u = lax.bitcast_convert_type(b, jnp.uint32)
    return jnp.where((u >> 31) != 0, ~u, u | jnp.uint32(0x80000000))


def _body(x_hbm, dc_hbm, o_hbm, bf_v, xa_v, xb_v, hist_v, ps_v, ck_v, ci_v,
          cx_v, sia, sib, soa, sob):
    wid = lax.axis_index("s") * 2 + lax.axis_index("c")
    iota = lax.iota(jnp.int32, _L)
    ones = jnp.ones((_L,), jnp.int32)
    zeros = jnp.zeros((_L,), jnp.int32)
    row0 = wid * _RPT

    in_a = pltpu.async_copy(x_hbm.at[row0], xa_v, sia)
    in_b = pltpu.async_copy(x_hbm.at[row0 + 1], xb_v, sib)

    pltpu.sync_copy(dc_hbm, bf_v)

    @plsc.parallel_loop(0, _N // _L, unroll=8)
    def _(i):
        sl = pl.ds(i * _L, _L)
        bf_v[sl] = jnp.exp(
            (jnp.float32(_TD) - bf_v[sl]) * jnp.float32(_BOOST_STRENGTH))

    def process_row(x_v, mid_hook):
        @plsc.parallel_loop(0, _HB // _L, unroll=8)
        def _(i):
            hist_v[pl.ds(i * _L, _L)] = zeros

        @plsc.parallel_loop(0, _N // _L, unroll=8)
        def _(i):
            sl = pl.ds(i * _L, _L)
            uk = _ukey(x_v[sl], bf_v[sl])
            bucket = (uk >> _RSHIFT).astype(jnp.int32)
            plsc.addupdate_scatter(hist_v, [bucket], ones)

        @plsc.parallel_loop(0, _NBLK)
        def _(t):
            acc = zeros
            for u in range(_CPB):
                acc = acc + hist_v[pl.ds(t * (_CPB * _L) + u * _L, _L)]
            ps_v[pl.ds(t * _L, _L)] = acc

        def b_body(t, carry):
            pfx, nblk, base = carry
            tot = jnp.sum(ps_v[pl.ds(t * _L, _L)])
            pfx = pfx + tot
            ok = pfx <= _NK
            return (pfx, nblk + ok.astype(jnp.int32),
                    jnp.where(ok, pfx, base))

        _pfx, tstar, base = lax.fori_loop(
            0, _NBLK, b_body, (jnp.int32(0), jnp.int32(0), jnp.int32(0)))

        def c_body(ci, carry):
            nb, cbv, tot = carry
            v = hist_v[pl.ds(tstar * (_CPB * _L) + ci * _L, _L)]
            s = plsc.cumsum(v) + tot
            mask = s <= _NK
            nb = nb + plsc.all_reduce_population_count(mask)
            cbv = jnp.maximum(cbv, jnp.where(mask, s, 0))
            return nb, cbv, jnp.max(s)

        nbv, cbv, _tot = lax.fori_loop(0, _CPB, c_body, (zeros, zeros, base))
        b_star = tstar * (_CPB * _L) + jnp.max(nbv)
        c_b0 = jnp.maximum(jnp.max(cbv), base)

        mid_hook()

        @plsc.parallel_loop(0, _N // _L, unroll=8, carry=zeros)
        def wptr_v(i, w):
            sl = pl.ds(i * _L, _L)
            xv = x_v[sl]
            uk = _ukey(xv, bf_v[sl])
            bucket = (uk >> _RSHIFT).astype(jnp.int32)
            win = bucket > b_star
            x_v[sl] = jnp.where(win, xv, jnp.float32(0.0))
            cand = bucket == b_star
            pos = plsc.cumsum(jnp.where(cand, 1, 0))
            dst = pos + (w - 1)
            rres = (uk & jnp.uint32(_RMASK)).astype(jnp.int32)
            plsc.store_scatter(ck_v, [dst], rres, mask=cand)
            plsc.store_scatter(ci_v, [dst], iota + i * _L, mask=cand)
            plsc.store_scatter(cx_v, [dst], xv, mask=cand)
            return w + plsc.all_reduce_population_count(cand)

        m = jnp.max(wptr_v)
        above = jnp.int32(_N) - c_b0 - m
        kp = jnp.int32(_K) - above
        nc = (m + _L - 1) // _L

        def bis_body(_, carry):
            lo, hi = carry
            mid = (lo + hi + 1) >> 1

            def cnt_body(j, acc):
                sl = pl.ds(j * _L, _L)
                mm = ((iota + j * _L) < m) & (ck_v[sl] >= mid)
                return acc + plsc.all_reduce_population_count(mm)

            cntv = lax.fori_loop(0, nc, cnt_body, zeros)
            pred = jnp.max(cntv) >= kp
            return jnp.where(pred, mid, lo), jnp.where(pred, hi, mid - 1)

        t_res, _hi = lax.fori_loop(
            0, _RSHIFT, bis_body, (jnp.int32(0), jnp.int32(_RMASK)))

        def f_body(j, cf):
            sl = pl.ds(j * _L, _L)
            wmask = ((iota + j * _L) < m) & (ck_v[sl] >= t_res)
            plsc.store_scatter(x_v, [ci_v[sl]], cx_v[sl], mask=wmask)
            return cf

        lax.fori_loop(0, nc, f_body, 0)

    def no_hook():
        return None

    in_a.wait()
    process_row(xa_v, no_hook)
    out_a = pltpu.async_copy(xa_v, o_hbm.at[row0], soa)

    in_b.wait()
    state = {}

    def hook_a():
        out_a.wait()
        state["in_a"] = pltpu.async_copy(x_hbm.at[row0 + 2], xa_v, sia)

    process_row(xb_v, hook_a)
    out_b = pltpu.async_copy(xb_v, o_hbm.at[row0 + 1], sob)

    state["in_a"].wait()

    def hook_b():
        out_b.wait()
        state["in_b"] = pltpu.async_copy(x_hbm.at[row0 + 3], xb_v, sib)

    process_row(xa_v, hook_b)
    out_a2 = pltpu.async_copy(xa_v, o_hbm.at[row0 + 2], soa)

    state["in_b"].wait()
    process_row(xb_v, no_hook)
    out_b2 = pltpu.async_copy(xb_v, o_hbm.at[row0 + 3], sob)

    out_a2.wait()
    out_b2.wait()


@jax.jit
def kernel(x, duty_cycles):
    run = pl.kernel(
        _body,
        out_type=jax.ShapeDtypeStruct((_B, _N), jnp.float32),
        mesh=plsc.VectorSubcoreMesh(core_axis_name="c", subcore_axis_name="s"),
        compiler_params=pltpu.CompilerParams(needs_layout_passes=False),
        scratch_types=[
            pltpu.VMEM((_N,), jnp.float32),
            pltpu.VMEM((_N,), jnp.float32),
            pltpu.VMEM((_N,), jnp.float32),
            pltpu.VMEM((_HB,), jnp.int32),
            pltpu.VMEM((_NBLK * _L,), jnp.int32),
            pltpu.VMEM((_CAP,), jnp.int32),
            pltpu.VMEM((_CAP,), jnp.int32),
            pltpu.VMEM((_CAP,), jnp.float32),
            pltpu.SemaphoreType.DMA,
            pltpu.SemaphoreType.DMA,
            pltpu.SemaphoreType.DMA,
            pltpu.SemaphoreType.DMA,
        ],
    )
    return run(x, duty_cycles)
